# baseline (speedup 1.0000x reference)
import functools

import jax
import jax.numpy as jnp
import numpy as np
from jax import lax
from jax.experimental import pallas as pl
from jax.experimental.pallas import tpu as pltpu
from jax.experimental.pallas import tpu_sc as plsc

_BIG = np.float32(3.0e38)
_K_SEL = 20
_CAND_W = 256
_TILE_W = 128


def _extract_lid_stats(read_chunk, nch, br, n_iter):
    def extract(it, carry):
        t, rem, ls, l1, l21, u = carry

        def body_min(cc, macc):
            ch = read_chunk(cc)
            cm = jnp.min(jnp.where(ch > t, ch, _BIG), axis=1, keepdims=True)
            return jnp.minimum(macc, cm)

        m = lax.fori_loop(0, nch, body_min,
                          jnp.full((br, 1), _BIG, jnp.float32))

        def body_cnt(cc, cacc):
            ch = read_chunk(cc)
            return cacc + jnp.sum(jnp.where(ch == m, 1.0, 0.0), axis=1,
                                  keepdims=True)

        c = lax.fori_loop(0, nch, body_cnt, jnp.zeros((br, 1), jnp.float32))

        take = jnp.minimum(c, rem)
        lg = jnp.log(jnp.maximum(m, 1e-12))
        ls = ls + take * lg
        l1 = jnp.where(it == 0, lg, l1)
        l21 = jnp.where(rem > 0, lg, l21)
        u = jnp.where(rem > 0, m, u)
        rem = rem - take
        return (m, rem, ls, l1, l21, u)

    init = (jnp.full((br, 1), -_BIG, jnp.float32),
            jnp.full((br, 1), float(n_iter), jnp.float32),
            jnp.zeros((br, 1), jnp.float32),
            jnp.zeros((br, 1), jnp.float32),
            jnp.zeros((br, 1), jnp.float32),
            jnp.zeros((br, 1), jnp.float32))
    _, _, ls, l1, l21, u = lax.fori_loop(0, n_iter, extract, init)
    return ls, l1, l21, u


def _dist_kernel(batch_ref, data_ref, d2_ref, u_ref, tm_ref, d2_scratch, *,
                 n_col_blocks, bc):
    j = pl.program_id(1)
    br = batch_ref.shape[0]

    bb = batch_ref[...]
    db = data_ref[...]
    bn = jnp.sum(bb * bb, axis=1, keepdims=True)
    dn = jnp.sum(db * db, axis=1)
    prod = lax.dot_general(bb.astype(jnp.bfloat16), db.astype(jnp.bfloat16),
                           (((1,), (1,)), ((), ())),
                           preferred_element_type=jnp.float32)
    d2 = bn + dn[None, :] - 2.0 * prod
    d2_ref[...] = d2
    d2_scratch[:, pl.ds(j * bc, bc)] = d2

    @pl.when(j == n_col_blocks - 1)
    def _():
        n = d2_scratch.shape[1]
        tm = jnp.concatenate(
            [jnp.min(d2_scratch[:, t * _TILE_W:(t + 1) * _TILE_W], axis=1,
                     keepdims=True)
             for t in range(n // _TILE_W)], axis=1)
        tm_ref[...] = tm
        _, _, _, u = _extract_lid_stats(lambda cc: tm, 1, br, _K_SEL + 1)
        u_ref[...] = u[:, 0]


def _sc_compact_kernel(d2_hbm, u_hbm, tm_hbm, out_hbm, rowbuf, u_v, tm_v,
                       idx_v, cand_v, sem0, sem1, *, rows_per_w, n, nc):
    wid = lax.axis_index("s") * nc + lax.axis_index("c")
    base = wid * rows_per_w
    n_tiles = n // _TILE_W

    pltpu.sync_copy(u_hbm.at[pl.ds(base, rows_per_w)],
                    u_v.at[pl.ds(0, rows_per_w)])
    pltpu.sync_copy(tm_hbm.at[pl.ds(base, rows_per_w)], tm_v)

    big16 = jnp.full((16,), _BIG, jnp.float32)
    iota16 = jnp.arange(16, dtype=jnp.int32)

    def prefill(i, _):
        cand_v[i // (_CAND_W // 16), pl.ds((i % (_CAND_W // 16)) * 16, 16)] = big16
        return 0
    lax.fori_loop(0, rows_per_w * (_CAND_W // 16), prefill, 0)

    sems = (sem0, sem1)

    def start_row_dma(r, par):
        pltpu.async_copy(d2_hbm.at[base + r], rowbuf.at[par], sems[par])

    def wait_row_dma(par):
        pltpu.make_async_copy(d2_hbm.at[0], rowbuf.at[par], sems[par]).wait()

    start_row_dma(0, 0)
    start_row_dma(1, 1)

    def process(r, par):
        u_s = u_v[pl.ds(r, 16)][0]
        ub = jnp.full((16,), u_s, jnp.float32)

        hcnt = jnp.int32(0)
        for g in range(n_tiles // 16):
            tmv = tm_v[r, pl.ds(g * 16, 16)]
            tmask = tmv <= ub
            keys = jnp.where(tmask, g * 16 + iota16, jnp.int32(n_tiles))
            ksi, _ = plsc.sort_key_val(keys, keys)
            idx_v[pl.ds(hcnt, 16)] = ksi
            pch = plsc.all_reduce_population_count(tmask)
            hcnt = jnp.minimum(hcnt + pch[0], jnp.int32(n_tiles))

        def tile_body(i, cnt):
            ti = idx_v[pl.ds(i, 16)][0]
            for g in range(_TILE_W // 16):
                v = rowbuf[par, pl.ds(ti * _TILE_W + g * 16, 16)]
                mask = v <= ub
                vk = jnp.where(mask, v, big16)
                ks, _ = plsc.sort_key_val(vk, vk)
                cand_v[r, pl.ds(cnt, 16)] = ks
                pc = plsc.all_reduce_population_count(mask)
                cnt = jnp.minimum(cnt + pc[0], _CAND_W - 16)
            return cnt

        lax.fori_loop(0, hcnt, tile_body, jnp.int32(0))

    def outer(g, _):
        for par in range(2):
            r = g * 2 + par
            wait_row_dma(par)
            process(r, par)

            @pl.when(r + 2 < rows_per_w)
            def _():
                start_row_dma(r + 2, par)
        return 0

    lax.fori_loop(0, rows_per_w // 2, outer, 0)

    pltpu.sync_copy(cand_v, out_hbm.at[pl.ds(base, rows_per_w)])


def _final_kernel(cand_ref, s_ref):
    br = cand_ref.shape[0]
    cw = 128
    nch = _CAND_W // cw
    ls, l1, l21, _ = _extract_lid_stats(
        lambda cc: cand_ref[:, pl.ds(cc * cw, cw)], nch, br, _K_SEL + 1)
    s = 0.5 * (ls - l1 - float(_K_SEL) * l21)
    s_ref[...] = s[:, 0]


def kernel(batch, data, k):
    q, d = batch.shape
    n, _ = data.shape
    br = min(256, q)
    bc = min(1024, n)
    tpb = bc // _TILE_W
    assert tpb <= 16

    grid = (q // br, n // bc)
    tm_w = grid[1] * 128
    d2, u, tm = pl.pallas_call(
        functools.partial(_dist_kernel, n_col_blocks=grid[1], bc=bc),
        grid=grid,
        in_specs=[
            pl.BlockSpec((br, d), lambda i, j: (i, 0)),
            pl.BlockSpec((bc, d), lambda i, j: (j, 0)),
        ],
        out_specs=[
            pl.BlockSpec((br, bc), lambda i, j: (i, j)),
            pl.BlockSpec((br,), lambda i, j: (i,)),
            pl.BlockSpec((br, n // _TILE_W), lambda i, j: (i, 0)),
        ],
        out_shape=[
            jax.ShapeDtypeStruct((q, n), jnp.float32),
            jax.ShapeDtypeStruct((q,), jnp.float32),
            jax.ShapeDtypeStruct((q, n // _TILE_W), jnp.float32),
        ],
        scratch_shapes=[pltpu.VMEM((br, n), jnp.float32)],
    )(batch, data)

    info = plsc.get_sparse_core_info()
    nc, ns = info.num_cores, info.num_subcores
    rows_per_w = q // (nc * ns)

    mesh = plsc.VectorSubcoreMesh(core_axis_name="c", subcore_axis_name="s")
    cand = pl.kernel(
        functools.partial(_sc_compact_kernel, rows_per_w=rows_per_w, n=n,
                          nc=nc),
        out_type=jax.ShapeDtypeStruct((q, _CAND_W), jnp.float32),
        mesh=mesh,
        compiler_params=pltpu.CompilerParams(needs_layout_passes=False),
        scratch_types=[
            pltpu.VMEM((2, n), jnp.float32),
            pltpu.VMEM((rows_per_w + 16,), jnp.float32),
            pltpu.VMEM((rows_per_w, n // _TILE_W), jnp.float32),
            pltpu.VMEM((n // _TILE_W + 16,), jnp.int32),
            pltpu.VMEM((rows_per_w, _CAND_W), jnp.float32),
            pltpu.SemaphoreType.DMA,
            pltpu.SemaphoreType.DMA,
        ],
    )(d2, u, tm)

    s = pl.pallas_call(
        _final_kernel,
        grid=(q // br,),
        in_specs=[pl.BlockSpec((br, _CAND_W), lambda i: (i, 0))],
        out_specs=pl.BlockSpec((br,), lambda i: (i,)),
        out_shape=jax.ShapeDtypeStruct((q,), jnp.float32),
    )(cand)

    return -jnp.asarray(k, jnp.float32) / (s + 1e-8)


# speedup vs baseline: 1.2557x; 1.2557x over previous
import functools

import jax
import jax.numpy as jnp
import numpy as np
from jax import lax
from jax.experimental import pallas as pl
from jax.experimental.pallas import tpu as pltpu
from jax.experimental.pallas import tpu_sc as plsc

_BIG = np.float32(3.0e38)
_K_SEL = 20
_CAND_W = 256
_TILE_W = 128


def _extract_lid_stats(read_chunk, nch, br, n_iter):
    def extract(it, carry):
        t, rem, ls, l1, l21, u = carry

        def body_min(cc, macc):
            ch = read_chunk(cc)
            cm = jnp.min(jnp.where(ch > t, ch, _BIG), axis=1, keepdims=True)
            return jnp.minimum(macc, cm)

        m = lax.fori_loop(0, nch, body_min,
                          jnp.full((br, 1), _BIG, jnp.float32))

        def body_cnt(cc, cacc):
            ch = read_chunk(cc)
            return cacc + jnp.sum(jnp.where(ch == m, 1.0, 0.0), axis=1,
                                  keepdims=True)

        c = lax.fori_loop(0, nch, body_cnt, jnp.zeros((br, 1), jnp.float32))

        take = jnp.minimum(c, rem)
        lg = jnp.log(jnp.maximum(m, 1e-12))
        ls = ls + take * lg
        l1 = jnp.where(it == 0, lg, l1)
        l21 = jnp.where(rem > 0, lg, l21)
        u = jnp.where(rem > 0, m, u)
        rem = rem - take
        return (m, rem, ls, l1, l21, u)

    init = (jnp.full((br, 1), -_BIG, jnp.float32),
            jnp.full((br, 1), float(n_iter), jnp.float32),
            jnp.zeros((br, 1), jnp.float32),
            jnp.zeros((br, 1), jnp.float32),
            jnp.zeros((br, 1), jnp.float32),
            jnp.zeros((br, 1), jnp.float32))
    _, _, ls, l1, l21, u = lax.fori_loop(0, n_iter, extract, init)
    return ls, l1, l21, u


def _dist_kernel(batch_ref, data_ref, d2_ref, u_ref, tm_ref, d2_scratch, *,
                 n_col_blocks, bc):
    j = pl.program_id(1)
    br = batch_ref.shape[0]

    bb = batch_ref[...]
    db = data_ref[...]
    bn = jnp.sum(bb * bb, axis=1, keepdims=True)
    dn = jnp.sum(db * db, axis=1)
    prod = lax.dot_general(bb, db, (((1,), (1,)), ((), ())),
                           preferred_element_type=jnp.float32)
    d2 = bn + dn[None, :] - 2.0 * prod
    d2_ref[...] = d2
    d2_scratch[:, pl.ds(j * bc, bc)] = d2

    @pl.when(j == n_col_blocks - 1)
    def _():
        n = d2_scratch.shape[1]
        tm = jnp.concatenate(
            [jnp.min(d2_scratch[:, t * _TILE_W:(t + 1) * _TILE_W], axis=1,
                     keepdims=True)
             for t in range(n // _TILE_W)], axis=1)
        tm_ref[...] = tm
        _, _, _, u = _extract_lid_stats(lambda cc: tm, 1, br, _K_SEL + 1)
        u_ref[...] = u[:, 0]


def _sc_compact_kernel(d2_hbm, u_hbm, tm_hbm, out_hbm, rowbuf, u_v, tm_v,
                       idx_v, cand_v, sem0, sem1, *, rows_per_w, n, nc):
    wid = lax.axis_index("s") * nc + lax.axis_index("c")
    base = wid * rows_per_w
    n_tiles = n // _TILE_W

    pltpu.sync_copy(u_hbm.at[pl.ds(base, rows_per_w)],
                    u_v.at[pl.ds(0, rows_per_w)])
    pltpu.sync_copy(tm_hbm.at[pl.ds(base, rows_per_w)], tm_v)

    big16 = jnp.full((16,), _BIG, jnp.float32)
    iota16 = jnp.arange(16, dtype=jnp.int32)

    def prefill(i, _):
        cand_v[i // (_CAND_W // 16), pl.ds((i % (_CAND_W // 16)) * 16, 16)] = big16
        return 0
    lax.fori_loop(0, rows_per_w * (_CAND_W // 16), prefill, 0)

    sems = (sem0, sem1)

    def start_row_dma(r, par):
        pltpu.async_copy(d2_hbm.at[base + r], rowbuf.at[par], sems[par])

    def wait_row_dma(par):
        pltpu.make_async_copy(d2_hbm.at[0], rowbuf.at[par], sems[par]).wait()

    start_row_dma(0, 0)
    start_row_dma(1, 1)

    def process(r, par):
        u_s = u_v[pl.ds(r, 16)][0]
        ub = jnp.full((16,), u_s, jnp.float32)

        hcnt = jnp.int32(0)
        for g in range(n_tiles // 16):
            tmv = tm_v[r, pl.ds(g * 16, 16)]
            tmask = tmv <= ub
            keys = jnp.where(tmask, g * 16 + iota16, jnp.int32(n_tiles))
            ksi, _ = plsc.sort_key_val(keys, keys)
            idx_v[pl.ds(hcnt, 16)] = ksi
            pch = plsc.all_reduce_population_count(tmask)
            hcnt = jnp.minimum(hcnt + pch[0], jnp.int32(n_tiles))

        def tile_body(i, cnt):
            ti = idx_v[pl.ds(i, 16)][0]
            for g in range(_TILE_W // 16):
                v = rowbuf[par, pl.ds(ti * _TILE_W + g * 16, 16)]
                mask = v <= ub
                vk = jnp.where(mask, v, big16)
                ks, _ = plsc.sort_key_val(vk, vk)
                cand_v[r, pl.ds(cnt, 16)] = ks
                pc = plsc.all_reduce_population_count(mask)
                cnt = jnp.minimum(cnt + pc[0], _CAND_W - 16)
            return cnt

        lax.fori_loop(0, hcnt, tile_body, jnp.int32(0))

    def outer(g, _):
        for par in range(2):
            r = g * 2 + par
            wait_row_dma(par)
            process(r, par)

            @pl.when(r + 2 < rows_per_w)
            def _():
                start_row_dma(r + 2, par)
        return 0

    lax.fori_loop(0, rows_per_w // 2, outer, 0)

    pltpu.sync_copy(cand_v, out_hbm.at[pl.ds(base, rows_per_w)])


def _final_kernel(cand_ref, s_ref):
    br = cand_ref.shape[0]
    cw = 128
    nch = _CAND_W // cw
    ls, l1, l21, _ = _extract_lid_stats(
        lambda cc: cand_ref[:, pl.ds(cc * cw, cw)], nch, br, _K_SEL + 1)
    s = 0.5 * (ls - l1 - float(_K_SEL) * l21)
    s_ref[...] = s[:, 0]


def kernel(batch, data, k):
    q, d = batch.shape
    n, _ = data.shape
    br = min(256, q)
    bc = min(1024, n)

    info = plsc.get_sparse_core_info()
    nc, ns = info.num_cores, info.num_subcores
    rows_per_w = br // (nc * ns)
    mesh = plsc.VectorSubcoreMesh(core_axis_name="c", subcore_axis_name="s")

    n_col_blocks = n // bc

    def stage1(batch_blk):
        return pl.pallas_call(
            functools.partial(_dist_kernel, n_col_blocks=n_col_blocks,
                              bc=bc),
            grid=(1, n_col_blocks),
            in_specs=[
                pl.BlockSpec((br, d), lambda i, j: (i, 0)),
                pl.BlockSpec((bc, d), lambda i, j: (j, 0)),
            ],
            out_specs=[
                pl.BlockSpec((br, bc), lambda i, j: (i, j)),
                pl.BlockSpec((br,), lambda i, j: (i,)),
                pl.BlockSpec((br, n // _TILE_W), lambda i, j: (i, 0)),
            ],
            out_shape=[
                jax.ShapeDtypeStruct((br, n), jnp.float32),
                jax.ShapeDtypeStruct((br,), jnp.float32),
                jax.ShapeDtypeStruct((br, n // _TILE_W), jnp.float32),
            ],
            scratch_shapes=[pltpu.VMEM((br, n), jnp.float32)],
        )(batch_blk, data)

    def stage2(d2, u, tm):
        return pl.kernel(
            functools.partial(_sc_compact_kernel, rows_per_w=rows_per_w,
                              n=n, nc=nc),
            out_type=jax.ShapeDtypeStruct((br, _CAND_W), jnp.float32),
            mesh=mesh,
            compiler_params=pltpu.CompilerParams(needs_layout_passes=False),
            scratch_types=[
                pltpu.VMEM((2, n), jnp.float32),
                pltpu.VMEM((rows_per_w + 16,), jnp.float32),
                pltpu.VMEM((rows_per_w, n // _TILE_W), jnp.float32),
                pltpu.VMEM((n // _TILE_W + 16,), jnp.int32),
                pltpu.VMEM((rows_per_w, _CAND_W), jnp.float32),
                pltpu.SemaphoreType.DMA,
                pltpu.SemaphoreType.DMA,
            ],
        )(d2, u, tm)

    cands = []
    for i in range(q // br):
        d2_i, u_i, tm_i = stage1(lax.slice_in_dim(batch, i * br,
                                                  (i + 1) * br, axis=0))
        cands.append(stage2(d2_i, u_i, tm_i))
    cand = jnp.concatenate(cands, axis=0)

    s = pl.pallas_call(
        _final_kernel,
        grid=(q // br,),
        in_specs=[pl.BlockSpec((br, _CAND_W), lambda i: (i, 0))],
        out_specs=pl.BlockSpec((br,), lambda i: (i,)),
        out_shape=jax.ShapeDtypeStruct((q,), jnp.float32),
    )(cand)

    return -jnp.asarray(k, jnp.float32) / (s + 1e-8)
